# Initial kernel scaffold; baseline (speedup 1.0000x reference)
import jax
import jax.numpy as jnp
from jax.experimental import pallas as pl


def kernel(X, edge_index, W, b):
    raise NotImplementedError("write your pallas kernel here")


# speedup vs baseline: 3.1224x; 3.1224x over previous
import functools

import jax
import jax.numpy as jnp
from jax import lax
from jax.experimental import pallas as pl
from jax.experimental.pallas import tpu as pltpu
from jax.experimental.pallas import tpu_sc as plsc

N_V = 10000
N_HE = 10000
D_IN = 128
D = 128
NP = 10240
NC = 2
NS = 16
NW = NC * NS
BS = 128
EB = 80
EP = NW * EB * BS


def _mm_body(x_ref, w_ref, b_ref, o_ref):
    o_ref[...] = (
        jnp.dot(x_ref[...], w_ref[...], preferred_element_type=jnp.float32)
        + b_ref[...]
    )


def _deg_body(dh_ref, oe_ref, ov_ref):
    oe_ref[...] = jnp.sum(dh_ref[:, 0, :], axis=0)
    ov_ref[...] = jnp.sum(dh_ref[:, 1, :], axis=0)


def _comb1_body(p_ref, d_ref, o_ref):
    deg = jnp.maximum(d_ref[...], 1.0)
    o_ref[...] = (p_ref[0] + p_ref[1]) / deg


def _comb2_body(p_ref, d_ref, o_ref):
    deg = jnp.maximum(d_ref[0:N_V], 1.0)
    y = (p_ref[0, 0:N_V] + p_ref[1, 0:N_V]) / deg
    o_ref[...] = jnp.where(y >= 0, y, 0.01 * y)


def _sc_pass_body(feat_hbm, gidx_hbm, sidx_hbm, *refs, with_hist):
    if with_hist:
        (outf_hbm, outd_hbm,
         gidx_v, sidx_v, buf, hist_s, hist_g, acc_sh) = refs
    else:
        outf_hbm, gidx_v, sidx_v, buf, acc_sh = refs
    c = lax.axis_index("c")
    s = lax.axis_index("s")
    rows_per_tile = NP // NS

    z16 = jnp.zeros((16,), jnp.float32)
    ones16 = jnp.ones((16,), jnp.float32)

    def zrow(i, carry):
        for k in range(D // 16):
            buf[i, pl.ds(k * 16, 16)] = z16
        return carry

    lax.fori_loop(0, BS, zrow, 0)
    if with_hist:
        def zhist(i, carry):
            hist_s[pl.ds(i * 16, 16)] = z16
            hist_g[pl.ds(i * 16, 16)] = z16
            return carry

        lax.fori_loop(0, NP // 16, zhist, 0)

    for k in range(rows_per_tile // BS):
        pltpu.sync_copy(buf, acc_sh.at[pl.ds(s * rows_per_tile + k * BS, BS)])
    plsc.subcore_barrier()

    def body(j, carry):
        pltpu.sync_copy(gidx_hbm.at[c, s, j], gidx_v)
        pltpu.sync_copy(sidx_hbm.at[c, s, j], sidx_v)
        pltpu.sync_copy(feat_hbm.at[gidx_v], buf)
        pltpu.sync_copy(buf, acc_sh.at[sidx_v], add=True)
        if with_hist:
            for k in range(BS // 16):
                si = sidx_v[pl.ds(k * 16, 16)]
                gi = gidx_v[pl.ds(k * 16, 16)]
                plsc.addupdate_scatter(hist_s, [si], ones16)
                plsc.addupdate_scatter(hist_g, [gi], ones16)
        return carry

    lax.fori_loop(0, EB, body, 0)
    plsc.subcore_barrier()

    pltpu.sync_copy(acc_sh.at[pl.ds(s * rows_per_tile, rows_per_tile)],
                    outf_hbm.at[c, pl.ds(s * rows_per_tile, rows_per_tile)])
    if with_hist:
        pltpu.sync_copy(hist_s, outd_hbm.at[c, s, 0])
        pltpu.sync_copy(hist_g, outd_hbm.at[c, s, 1])


_MESH = plsc.VectorSubcoreMesh(core_axis_name="c", subcore_axis_name="s")

_sc_pass1 = functools.partial(
    pl.kernel,
    mesh=_MESH,
    compiler_params=pltpu.CompilerParams(needs_layout_passes=False),
    out_type=[
        jax.ShapeDtypeStruct((NC, NP, D), jnp.float32),
        jax.ShapeDtypeStruct((NC, NS, 2, NP), jnp.float32),
    ],
    scratch_types=[
        pltpu.VMEM((BS,), jnp.int32),
        pltpu.VMEM((BS,), jnp.int32),
        pltpu.VMEM((BS, D), jnp.float32),
        pltpu.VMEM((NP,), jnp.float32),
        pltpu.VMEM((NP,), jnp.float32),
        pltpu.VMEM_SHARED((NP, D), jnp.float32),
    ],
)(functools.partial(_sc_pass_body, with_hist=True))

_sc_pass2 = functools.partial(
    pl.kernel,
    mesh=_MESH,
    compiler_params=pltpu.CompilerParams(needs_layout_passes=False),
    out_type=jax.ShapeDtypeStruct((NC, NP, D), jnp.float32),
    scratch_types=[
        pltpu.VMEM((BS,), jnp.int32),
        pltpu.VMEM((BS,), jnp.int32),
        pltpu.VMEM((BS, D), jnp.float32),
        pltpu.VMEM_SHARED((NP, D), jnp.float32),
    ],
)(functools.partial(_sc_pass_body, with_hist=False))


def kernel(X, edge_index, W, b):
    X = X.astype(jnp.float32)
    W = W.astype(jnp.float32)
    b = b.astype(jnp.float32)

    X_pad = jnp.zeros((NP, D_IN), jnp.float32).at[:N_V].set(X)
    Xt = pl.pallas_call(
        _mm_body,
        out_shape=jax.ShapeDtypeStruct((NP, D), jnp.float32),
    )(X_pad, W, b[None, :])

    vid = edge_index[0].astype(jnp.int32)
    eid = edge_index[1].astype(jnp.int32)
    e_inc = vid.shape[0]
    pad = jnp.full((EP - e_inc,), NP - 1, jnp.int32)
    vid_b = jnp.concatenate([vid, pad]).reshape(NC, NS, EB, BS)
    eid_b = jnp.concatenate([eid, pad]).reshape(NC, NS, EB, BS)

    p1, dh = _sc_pass1(Xt, vid_b, eid_b)
    deg_e, deg_v = pl.pallas_call(
        _deg_body,
        out_shape=[
            jax.ShapeDtypeStruct((NP,), jnp.float32),
            jax.ShapeDtypeStruct((NP,), jnp.float32),
        ],
    )(dh.reshape(NW, 2, NP))

    e_feat = pl.pallas_call(
        _comb1_body,
        out_shape=jax.ShapeDtypeStruct((NP, D), jnp.float32),
    )(p1, deg_e.reshape(NP, 1))

    p2 = _sc_pass2(e_feat, eid_b, vid_b)
    out = pl.pallas_call(
        _comb2_body,
        out_shape=jax.ShapeDtypeStruct((N_V, D), jnp.float32),
    )(p2, deg_v.reshape(NP, 1))
    return out
